# Initial kernel scaffold; baseline (speedup 1.0000x reference)
import jax
import jax.numpy as jnp
from jax.experimental import pallas as pl


def kernel(nf, ef, u, edge_index, params):
    raise NotImplementedError("write your pallas kernel here")


# speedup vs baseline: 3.6681x; 3.6681x over previous
import functools

import jax
import jax.numpy as jnp
from jax import lax
from jax.experimental import pallas as pl
from jax.experimental.pallas import tpu as pltpu
from jax.experimental.pallas import tpu_sc as plsc

N = 10000
NP = 10240
E = 320000
F32 = jnp.float32

NC = 2
NS = 16
NW = NC * NS
EW = E // NW
ROWS_T = NP // NS

_MESH = dict(core_axis_name="c", subcore_axis_name="s")


def _relu(x):
    return jnp.maximum(x, 0.0)


def _dot(a, b):
    return jnp.dot(a, b, preferred_element_type=F32)


def _sc_gather_add(ps, pd, src, dst):
    CH = 400
    NCH = EW // CH

    def body(ps_h, pd_h, src_h, dst_h, out_h, si_v, di_v, ra_v, rb_v, sa, sb):
        cid = lax.axis_index("c")
        sid = lax.axis_index("s")
        base = (sid * NC + cid) * EW

        def chunk(ci, carry):
            off = base + ci * CH
            pltpu.sync_copy(src_h.at[pl.ds(off, CH)], si_v)
            pltpu.sync_copy(dst_h.at[pl.ds(off, CH)], di_v)
            ca = pltpu.async_copy(ps_h.at[si_v], ra_v, sa)
            cb = pltpu.async_copy(pd_h.at[di_v], rb_v, sb)
            ca.wait()
            cb.wait()

            def addrow(r, c2):
                for j in range(4):
                    sl = pl.ds(16 * j, 16)
                    ra_v[r, sl] = ra_v[r, sl] + rb_v[r, sl]
                return c2

            lax.fori_loop(0, CH, addrow, 0)
            pltpu.sync_copy(ra_v, out_h.at[pl.ds(off, CH)])
            return carry

        lax.fori_loop(0, NCH, chunk, 0)

    f = pl.kernel(
        body,
        out_type=jax.ShapeDtypeStruct((E, 64), F32),
        mesh=plsc.VectorSubcoreMesh(**_MESH),
        compiler_params=pltpu.CompilerParams(
            use_tc_tiling_on_sc=False, needs_layout_passes=False),
        scratch_types=[
            pltpu.VMEM((CH,), jnp.int32),
            pltpu.VMEM((CH,), jnp.int32),
            pltpu.VMEM((CH, 64), F32),
            pltpu.VMEM((CH, 64), F32),
            pltpu.SemaphoreType.DMA,
            pltpu.SemaphoreType.DMA,
        ],
    )
    return f(ps, pd, src, dst)


def _sc_scatter_sum(vals, dst, with_counts):
    CH = 1000
    NCH = EW // CH
    VD = 32

    out_type = [jax.ShapeDtypeStruct((NC, NP, VD), F32)]
    scratch = [
        pltpu.VMEM((CH,), jnp.int32),
        pltpu.VMEM((CH, VD), F32),
        pltpu.VMEM((ROWS_T, VD), F32),
        pltpu.VMEM_SHARED((NP, VD), F32),
    ]
    if with_counts:
        out_type.append(jax.ShapeDtypeStruct((NC, NP), F32))
        scratch += [
            pltpu.VMEM((CH,), F32),
            pltpu.VMEM((ROWS_T,), F32),
            pltpu.VMEM_SHARED((NP,), F32),
        ]

    def body(vals_h, dst_h, *refs):
        if with_counts:
            out_h, cnt_h, di_v, rv_v, zb_v, acc_sh, ones_v, z1_v, acc1_sh = refs
        else:
            out_h, di_v, rv_v, zb_v, acc_sh = refs
        cid = lax.axis_index("c")
        sid = lax.axis_index("s")
        base = (sid * NC + cid) * EW
        sl = pl.ds(sid * ROWS_T, ROWS_T)

        def zrow(r, c):
            zb_v[r, pl.ds(0, 16)] = jnp.zeros((16,), F32)
            zb_v[r, pl.ds(16, 16)] = jnp.zeros((16,), F32)
            return c

        lax.fori_loop(0, ROWS_T, zrow, 0)
        pltpu.sync_copy(zb_v, acc_sh.at[sl])
        if with_counts:
            def orow(g, c):
                ones_v[pl.ds(g * 16, 16)] = jnp.ones((16,), F32)
                return c

            lax.fori_loop(0, CH // 16, orow, 0)

            def z1row(g, c):
                z1_v[pl.ds(g * 16, 16)] = jnp.zeros((16,), F32)
                return c

            lax.fori_loop(0, ROWS_T // 16, z1row, 0)
            pltpu.sync_copy(z1_v, acc1_sh.at[sl])
        plsc.subcore_barrier()

        def chunk(ci, carry):
            off = base + ci * CH
            pltpu.sync_copy(dst_h.at[pl.ds(off, CH)], di_v)
            pltpu.sync_copy(vals_h.at[pl.ds(off, CH)], rv_v)
            pltpu.sync_copy(rv_v, acc_sh.at[di_v], add=True)
            if with_counts:
                pltpu.sync_copy(ones_v, acc1_sh.at[di_v], add=True)
            return carry

        lax.fori_loop(0, NCH, chunk, 0)
        plsc.subcore_barrier()
        pltpu.sync_copy(acc_sh.at[sl], out_h.at[cid, sl])
        if with_counts:
            pltpu.sync_copy(acc1_sh.at[sl], cnt_h.at[cid, sl])

    f = pl.kernel(
        body,
        out_type=tuple(out_type),
        mesh=plsc.VectorSubcoreMesh(**_MESH),
        compiler_params=pltpu.CompilerParams(
            use_tc_tiling_on_sc=False, needs_layout_passes=False),
        scratch_types=scratch,
    )
    return f(vals, dst)


def _sc_segmax(logits, dst):
    CH = 400
    NCH = EW // CH
    NEG = -1e30

    def body(lg_h, dst_h, out_h, di_v, lg_v, acc_v, ra_v, rb_v, slots_sh):
        cid = lax.axis_index("c")
        sid = lax.axis_index("s")
        base = (sid * NC + cid) * EW

        def initr(g, c):
            acc_v[pl.ds(g * 16, 16)] = jnp.full((16,), NEG, F32)
            return c

        lax.fori_loop(0, NP // 16, initr, 0)

        def chunk(ci, carry):
            off = base + ci * CH
            pltpu.sync_copy(dst_h.at[pl.ds(off, CH)], di_v)
            pltpu.sync_copy(lg_h.at[pl.ds(off, CH)], lg_v)

            def grp(g, c):
                s = pl.ds(g * 16, 16)
                idx = di_v[s]
                lg = lg_v[s]

                def rnd(r, c2):
                    cur = plsc.load_gather(acc_v, [idx])
                    plsc.store_scatter(acc_v, [idx], jnp.maximum(cur, lg))
                    return c2

                lax.fori_loop(0, 8, rnd, 0)
                return c

            lax.fori_loop(0, CH // 16, grp, 0)
            return carry

        lax.fori_loop(0, NCH, chunk, 0)
        pltpu.sync_copy(acc_v, slots_sh.at[sid])
        plsc.subcore_barrier()
        sl = pl.ds(sid * ROWS_T, ROWS_T)
        pltpu.sync_copy(slots_sh.at[0, sl], ra_v)
        for t in range(1, NS):
            pltpu.sync_copy(slots_sh.at[t, sl], rb_v)

            def mrow(g, c):
                s = pl.ds(g * 16, 16)
                ra_v[s] = jnp.maximum(ra_v[s], rb_v[s])
                return c

            lax.fori_loop(0, ROWS_T // 16, mrow, 0)
        pltpu.sync_copy(ra_v, out_h.at[cid, sl])

    f = pl.kernel(
        body,
        out_type=jax.ShapeDtypeStruct((NC, NP), F32),
        mesh=plsc.VectorSubcoreMesh(**_MESH),
        compiler_params=pltpu.CompilerParams(
            use_tc_tiling_on_sc=False, needs_layout_passes=False),
        scratch_types=[
            pltpu.VMEM((CH,), jnp.int32),
            pltpu.VMEM((CH,), F32),
            pltpu.VMEM((NP,), F32),
            pltpu.VMEM((ROWS_T,), F32),
            pltpu.VMEM((ROWS_T,), F32),
            pltpu.VMEM_SHARED((NS, NP), F32),
        ],
    )
    return f(logits, dst)


def _sc_att_ex(logits, dst, smax2):
    CH = 400
    NCH = EW // CH

    def body(lg_h, dst_h, sm2_h, ex_h, den_h,
             di_v, lg_v, ex_v, sma_v, smb_v, z1_v, den_sh):
        cid = lax.axis_index("c")
        sid = lax.axis_index("s")
        base = (sid * NC + cid) * EW
        sl = pl.ds(sid * ROWS_T, ROWS_T)

        pltpu.sync_copy(sm2_h.at[0], sma_v)
        pltpu.sync_copy(sm2_h.at[1], smb_v)

        def cmb(g, c):
            s = pl.ds(g * 16, 16)
            sma_v[s] = jnp.maximum(sma_v[s], smb_v[s])
            return c

        lax.fori_loop(0, NP // 16, cmb, 0)

        def z1row(g, c):
            z1_v[pl.ds(g * 16, 16)] = jnp.zeros((16,), F32)
            return c

        lax.fori_loop(0, ROWS_T // 16, z1row, 0)
        pltpu.sync_copy(z1_v, den_sh.at[sl])
        plsc.subcore_barrier()

        def chunk(ci, carry):
            off = base + ci * CH
            pltpu.sync_copy(dst_h.at[pl.ds(off, CH)], di_v)
            pltpu.sync_copy(lg_h.at[pl.ds(off, CH)], lg_v)

            def grp(g, c):
                s = pl.ds(g * 16, 16)
                sm = plsc.load_gather(sma_v, [di_v[s]])
                ex_v[s] = jnp.exp(lg_v[s] - sm)
                return c

            lax.fori_loop(0, CH // 16, grp, 0)
            pltpu.sync_copy(ex_v, ex_h.at[pl.ds(off, CH)])
            pltpu.sync_copy(ex_v, den_sh.at[di_v], add=True)
            return carry

        lax.fori_loop(0, NCH, chunk, 0)
        plsc.subcore_barrier()
        pltpu.sync_copy(den_sh.at[sl], den_h.at[cid, sl])

    f = pl.kernel(
        body,
        out_type=(jax.ShapeDtypeStruct((E,), F32),
                  jax.ShapeDtypeStruct((NC, NP), F32)),
        mesh=plsc.VectorSubcoreMesh(**_MESH),
        compiler_params=pltpu.CompilerParams(
            use_tc_tiling_on_sc=False, needs_layout_passes=False),
        scratch_types=[
            pltpu.VMEM((CH,), jnp.int32),
            pltpu.VMEM((CH,), F32),
            pltpu.VMEM((CH,), F32),
            pltpu.VMEM((NP,), F32),
            pltpu.VMEM((NP,), F32),
            pltpu.VMEM((ROWS_T,), F32),
            pltpu.VMEM_SHARED((NP,), F32),
        ],
    )
    return f(logits, dst, smax2)


def _tc_prep0(cnf, ws, wd, u, wu, b1):
    def body(cnf_r, ws_r, wd_r, u_r, wu_r, b1_r, ps_r, pd_r):
        x = cnf_r[...]
        ut = _dot(u_r[...], wu_r[...]) + b1_r[...]
        ps_r[...] = _dot(x, ws_r[...]) + ut
        pd_r[...] = _dot(x, wd_r[...])

    return pl.pallas_call(
        body,
        out_shape=(jax.ShapeDtypeStruct((NP, 64), F32),
                   jax.ShapeDtypeStruct((NP, 64), F32)),
    )(cnf, ws, wd, u, wu, b1)


def _tc_globprep(cu, esum, nsum, gm_parts, nnf, em_parts, resid_u):
    (wgu, wge, wgn, b1g, w2g, b2g, w3g, b3g) = gm_parts
    (ws, wd, wu, b1e) = em_parts

    def body(cu_r, es_r, ns_r, wgu_r, wge_r, wgn_r, b1g_r, w2g_r, b2g_r,
             w3g_r, b3g_r, nnf_r, ws_r, wd_r, wu_r, b1e_r,
             u_out, ps_r, pd_r):
        cu_ = cu_r[...]
        em = es_r[...] * (1.0 / E)
        nm = ns_r[...] * (1.0 / N)
        h = _relu(_dot(cu_, wgu_r[...]) + _dot(em, wge_r[...])
                  + _dot(nm, wgn_r[...]) + b1g_r[...])
        h = _relu(_dot(h, w2g_r[...]) + b2g_r[...])
        h = _relu(_dot(h, w3g_r[...]) + b3g_r[...])
        nu = h + cu_ if resid_u else h
        u_out[...] = nu
        x = nnf_r[...]
        ut = _dot(nu, wu_r[...]) + b1e_r[...]
        ps_r[...] = _dot(x, ws_r[...]) + ut
        pd_r[...] = _dot(x, wd_r[...])

    return pl.pallas_call(
        body,
        out_shape=(jax.ShapeDtypeStruct((1, 32), F32),
                   jax.ShapeDtypeStruct((NP, 64), F32),
                   jax.ShapeDtypeStruct((NP, 64), F32)),
    )(cu, esum, nsum, wgu, wge, wgn, b1g, w2g, b2g, w3g, b3g,
      nnf, ws, wd, wu, b1e)


def _tc_edge(s1, cef, we, w2, b2, w3, b3, residual, att_w=None):
    BE = 2000
    G = E // BE
    ei = cef.shape[1]
    att = att_w is not None

    def body(*refs):
        if att:
            (s1_r, cef_r, we_r, w2_r, b2_r, w3_r, b3_r, aw_r,
             out_r, lg_r) = refs
        else:
            (s1_r, cef_r, we_r, w2_r, b2_r, w3_r, b3_r,
             out_r, es_r) = refs
        pid = pl.program_id(0)
        h = _relu(s1_r[...] + _dot(cef_r[...], we_r[...]))
        h = _relu(_dot(h, w2_r[...]) + b2_r[...])
        o = _relu(_dot(h, w3_r[...]) + b3_r[...])
        if att:
            out_r[...] = o
            lg_r[...] = _dot(o, aw_r[...])
        else:
            out_r[...] = o + cef_r[...] if residual else o
            ps = jnp.sum(o, axis=0, keepdims=True)

            @pl.when(pid == 0)
            def _():
                es_r[...] = ps

            @pl.when(pid != 0)
            def _():
                es_r[...] = es_r[...] + ps

    full = lambda a: pl.BlockSpec(a.shape, lambda i: (0,) * a.ndim)
    in_specs = [
        pl.BlockSpec((BE, 64), lambda i: (i, 0)),
        pl.BlockSpec((BE, ei), lambda i: (i, 0)),
        full(we), full(w2), full(b2), full(w3), full(b3),
    ]
    args = [s1, cef, we, w2, b2, w3, b3]
    if att:
        in_specs.append(full(att_w))
        args.append(att_w)
        out_shape = (jax.ShapeDtypeStruct((E, 32), F32),
                     jax.ShapeDtypeStruct((E, 1), F32))
        out_specs = (pl.BlockSpec((BE, 32), lambda i: (i, 0)),
                     pl.BlockSpec((BE, 1), lambda i: (i, 0)))
    else:
        out_shape = (jax.ShapeDtypeStruct((E, 32), F32),
                     jax.ShapeDtypeStruct((1, 32), F32))
        out_specs = (pl.BlockSpec((BE, 32), lambda i: (i, 0)),
                     pl.BlockSpec((1, 32), lambda i: (0, 0)))

    return pl.pallas_call(
        body, grid=(G,), in_specs=in_specs, out_specs=out_specs,
        out_shape=out_shape,
    )(*args)


def _tc_scale(pre, ex):
    BE = 2000
    G = E // BE

    def body(p_r, e_r, o_r):
        o_r[...] = p_r[...] * e_r[...]

    return pl.pallas_call(
        body, grid=(G,),
        in_specs=[pl.BlockSpec((BE, 32), lambda i: (i, 0)),
                  pl.BlockSpec((BE, 1), lambda i: (i, 0))],
        out_specs=pl.BlockSpec((BE, 32), lambda i: (i, 0)),
        out_shape=jax.ShapeDtypeStruct((E, 32), F32),
    )(pre, ex)


def _tc_node_mean(cnf, aggS, cnt2, t_arr, u, nm_parts, residual, want_t):
    (wc, wa, wu, b1, w2, b2, w3, b3) = nm_parts
    with_t = t_arr is not None

    def body(*refs):
        i = 0
        cnf_r = refs[i]; i += 1
        agg_r = refs[i]; i += 1
        cnt_r = refs[i]; i += 1
        if with_t:
            t_r = refs[i]; i += 1
        u_r = refs[i]; i += 1
        wc_r, wa_r, wu_r, b1_r, w2_r, b2_r, w3_r, b3_r = refs[i:i + 8]
        i += 8
        nf_out = refs[i]; i += 1
        ns_out = refs[i]; i += 1
        if want_t:
            t_out = refs[i]

        ssum = agg_r[0] + agg_r[1]
        pre = ssum - t_r[...] if with_t else ssum
        cnt = jnp.maximum(cnt_r[0] + cnt_r[1], 1.0)[:, None]
        agg = pre / cnt
        x = cnf_r[...]
        h = _relu(_dot(x, wc_r[...]) + _dot(agg, wa_r[...])
                  + _dot(u_r[...], wu_r[...]) + b1_r[...])
        h = _relu(_dot(h, w2_r[...]) + b2_r[...])
        o = _relu(_dot(h, w3_r[...]) + b3_r[...])
        rows = lax.broadcasted_iota(jnp.int32, (NP, 1), 0)
        ns_out[...] = jnp.sum(jnp.where(rows < N, o, 0.0), axis=0,
                              keepdims=True)
        nf_out[...] = o + x if residual else o
        if want_t:
            t_out[...] = ssum

    args = [cnf, aggS, cnt2]
    if with_t:
        args.append(t_arr)
    args += [u, *nm_parts]
    out_shape = [jax.ShapeDtypeStruct((NP, 32), F32),
                 jax.ShapeDtypeStruct((1, 32), F32)]
    if want_t:
        out_shape.append(jax.ShapeDtypeStruct((NP, 32), F32))

    return pl.pallas_call(body, out_shape=tuple(out_shape))(*args)


def _tc_node_att(cnf, aggE, den2, u, nm_parts):
    (wc, wa, wu, b1, w2, b2, w3, b3) = nm_parts

    def body(cnf_r, agg_r, den_r, u_r, wc_r, wa_r, wu_r, b1_r,
             w2_r, b2_r, w3_r, b3_r, nf_out):
        den = (den_r[0] + den_r[1] + 1e-16)[:, None]
        agg = (agg_r[0] + agg_r[1]) / den
        x = cnf_r[...]
        h = _relu(_dot(x, wc_r[...]) + _dot(agg, wa_r[...])
                  + _dot(u_r[...], wu_r[...]) + b1_r[...])
        h = _relu(_dot(h, w2_r[...]) + b2_r[...])
        o = _relu(_dot(h, w3_r[...]) + b3_r[...])
        nf_out[...] = o + x

    return pl.pallas_call(
        body, out_shape=jax.ShapeDtypeStruct((NP, 32), F32),
    )(cnf, aggE, den2, u, *nm_parts)


def _tc_reg(cnf, reg_params):
    ws = []
    for (w, b) in reg_params:
        ws += [w, b.reshape(1, -1)]

    def body(cnf_r, w1_r, b1_r, w2_r, b2_r, w3_r, b3_r, w4_r, b4_r, out_r):
        x = cnf_r[...]
        x = _relu(_dot(x, w1_r[...]) + b1_r[...])
        x = _relu(_dot(x, w2_r[...]) + b2_r[...])
        x = _relu(_dot(x, w3_r[...]) + b3_r[...])
        x = _relu(_dot(x, w4_r[...]) + b4_r[...])
        out_r[...] = jnp.clip(x, 0.0, 1.0)

    return pl.pallas_call(
        body, out_shape=jax.ShapeDtypeStruct((NP, 1), F32),
    )(cnf, *ws)


def _split_em(layer, ei, ni):
    (w1, b1), (w2, b2), (w3, b3) = layer['em']
    we = w1[:ei]
    ws = w1[ei:ei + ni]
    wd = w1[ei + ni:ei + 2 * ni]
    wu = w1[ei + 2 * ni:]
    return we, ws, wd, wu, b1.reshape(1, -1), w2, b2.reshape(1, -1), w3, \
        b3.reshape(1, -1)


def _split_nm(layer, ni):
    (w1, b1), (w2, b2), (w3, b3) = layer['nm']
    wc = w1[:ni]
    wa = w1[ni:ni + 32]
    wu = w1[ni + 32:]
    return (wc, wa, wu, b1.reshape(1, -1), w2, b2.reshape(1, -1), w3,
            b3.reshape(1, -1))


def _split_gm(layer, gi):
    (w1, b1), (w2, b2), (w3, b3) = layer['gm']
    wgu = w1[:gi]
    wge = w1[gi:gi + 32]
    wgn = w1[gi + 32:]
    return (wgu, wge, wgn, b1.reshape(1, -1), w2, b2.reshape(1, -1), w3,
            b3.reshape(1, -1))


def kernel(nf, ef, u, edge_index, params):
    layers = params['layers']
    src = edge_index[0]
    dst = edge_index[1]
    nf_p = jnp.pad(nf, ((0, NP - N), (0, 0)))

    ein = [16, 32, 32, 32]
    nin = [128, 32, 32, 32]

    we0, ws0, wd0, wu0, b10, w20, b20, w30, b30 = _split_em(layers[0], 16, 128)
    ps, pd = _tc_prep0(nf_p, ws0, wd0, u, wu0, b10)
    s1 = _sc_gather_add(ps, pd, src, dst)
    nef, esum = _tc_edge(s1, ef, we0, w20, b20, w30, b30, residual=False)
    aggS, cnt2 = _sc_scatter_sum(nef, dst, with_counts=True)
    nnf, nsum, t_arr = _tc_node_mean(
        nf_p, aggS, cnt2, None, u, _split_nm(layers[0], 128),
        residual=False, want_t=True)
    cu, ps, pd = _tc_globprep(
        u, esum, nsum, _split_gm(layers[0], 16), nnf,
        _split_em(layers[1], 32, 32)[1:4] + (_split_em(layers[1], 32, 32)[4],),
        resid_u=False)

    for i in (1, 2):
        wei, _, _, _, _, w2i, b2i, w3i, b3i = _split_em(layers[i], 32, 32)
        s1 = _sc_gather_add(ps, pd, src, dst)
        nef_new, esum = _tc_edge(s1, nef, wei, w2i, b2i, w3i, b3i,
                                 residual=True)
        aggS = _sc_scatter_sum(nef_new, dst, with_counts=False)[0]
        node_out = _tc_node_mean(
            nnf, aggS, cnt2, t_arr, cu, _split_nm(layers[i], 32),
            residual=True, want_t=(i < 2))
        nnf_new, nsum = node_out[0], node_out[1]
        emn = _split_em(layers[i + 1], 32, 32)
        cu, ps, pd = _tc_globprep(
            cu, esum, nsum, _split_gm(layers[i], 32), nnf_new,
            emn[1:4] + (emn[4],), resid_u=True)
        nef = nef_new
        nnf = nnf_new
        if i < 2:
            t_arr = node_out[2]

    we3, _, _, _, _, w23, b23, w33, b33 = _split_em(layers[3], 32, 32)
    s1 = _sc_gather_add(ps, pd, src, dst)
    pre3, logits = _tc_edge(s1, nef, we3, w23, b23, w33, b33,
                            residual=False, att_w=layers[3]['att'])
    lg_flat = logits.reshape(E)
    smax2 = _sc_segmax(lg_flat, dst)
    ex, den2 = _sc_att_ex(lg_flat, dst, smax2)
    wef = _tc_scale(pre3, ex.reshape(E, 1))
    aggE = _sc_scatter_sum(wef, dst, with_counts=False)[0]
    nnf = _tc_node_att(nnf, aggE, den2, cu, _split_nm(layers[3], 32))

    pred = _tc_reg(nnf, params['reg'])
    return pred[:N]
